# baseline (speedup 1.0000x reference)
import functools

import jax
import jax.numpy as jnp
from jax import lax
from jax.experimental import pallas as pl
from jax.experimental.pallas import tpu as pltpu
from jax.experimental.pallas import tpu_sc as plsc

_B, _N, _D = 4, 4096, 2048
_NW = 32
_NS = 16
_ROWS_PER_W = _B * _N // _NW
_C = 4
_NCHUNKS = _ROWS_PER_W // _C
_NBUF = 8
_GDEPTH = 7
_NSLOT = 4


@jax.jit
def _gather(x_flat, idx3):
    mesh = plsc.VectorSubcoreMesh(core_axis_name="c", subcore_axis_name="s")

    @functools.partial(
        pl.kernel,
        mesh=mesh,
        out_type=jax.ShapeDtypeStruct((_B * _N, _D), jnp.float32),
        scratch_types=[
            pltpu.VMEM((_NCHUNKS, _C), jnp.int32),
            pltpu.VMEM((_NBUF, _C, _D), jnp.float32),
            pltpu.VMEM_SHARED((_NS, _NSLOT, _C, _D), jnp.float32),
        ]
        + [pltpu.SemaphoreType.DMA] * (_NBUF + 2 * _NSLOT),
    )
    def k(x_hbm, idx_hbm, out_hbm, idx_v, buf, shared, *sems):
        gsem = sems[:_NBUF]
        csem = sems[_NBUF : _NBUF + _NSLOT]
        wsem = sems[_NBUF + _NSLOT :]
        info = plsc.get_sparse_core_info()
        sid = lax.axis_index("s")
        wid = sid * info.num_cores + lax.axis_index("c")
        pltpu.sync_copy(idx_hbm.at[wid], idx_v)
        row_base = wid * _ROWS_PER_W

        def start_gather(c, b):
            pltpu.make_async_copy(x_hbm.at[idx_v.at[c]], buf.at[b], gsem[b]).start()

        def wait_gather(b):
            pltpu.make_async_copy(x_hbm.at[pl.ds(0, _C)], buf.at[b], gsem[b]).wait()

        def start_copy(b, s):
            pltpu.make_async_copy(buf.at[b], shared.at[sid].at[s], csem[s]).start()

        def wait_copy(b, s):
            pltpu.make_async_copy(buf.at[b], shared.at[sid].at[s], csem[s]).wait()

        def start_write(c, s):
            pltpu.make_async_copy(
                shared.at[sid].at[s], out_hbm.at[pl.ds(row_base + c * _C, _C)], wsem[s]
            ).start()

        def wait_write(s):
            pltpu.make_async_copy(
                shared.at[sid].at[s], out_hbm.at[pl.ds(row_base, _C)], wsem[s]
            ).wait()

        for p in range(_GDEPTH):
            start_gather(p, p)

        def body(g, carry):
            for b in range(_NBUF):
                c = _NBUF * g + b
                s = b % _NSLOT
                wait_gather(b)

                @pl.when(c >= _NSLOT)
                def _():
                    wait_write(s)

                start_copy(b, s)
                wait_copy(b, s)
                start_write(c, s)

                @pl.when(c + _GDEPTH < _NCHUNKS)
                def _():
                    start_gather(c + _GDEPTH, (b + _GDEPTH) % _NBUF)
            return carry

        lax.fori_loop(0, _NCHUNKS // _NBUF, body, 0)
        for s in range(_NSLOT):
            wait_write(s)

    return k(x_flat, idx3)


def _perm_indices(B, N):
    base_key = jax.random.key(42)

    def one(i):
        return jax.random.permutation(jax.random.fold_in(base_key, i), N)

    perm = jax.vmap(one)(jnp.arange(B))
    flat = perm.astype(jnp.int32) + (jnp.arange(B, dtype=jnp.int32) * N)[:, None]
    return flat.reshape(_NW, _NCHUNKS, _C)


def kernel(x):
    B, N, D = x.shape
    idx3 = _perm_indices(B, N)
    out = _gather(x.reshape(B * N, D), idx3)
    return out.reshape(B, N, D)


# speedup vs baseline: 1.0084x; 1.0084x over previous
import functools

import jax
import jax.numpy as jnp
from jax import lax
from jax.experimental import pallas as pl
from jax.experimental.pallas import tpu as pltpu
from jax.experimental.pallas import tpu_sc as plsc

_B, _N, _D = 4, 4096, 2048
_NW = 32
_NS = 16
_ROWS_PER_W = _B * _N // _NW
_C = 8
_NCHUNKS = _ROWS_PER_W // _C
_NBUF = 4
_GDEPTH = 3
_NSLOT = 2


@jax.jit
def _gather(x_flat, idx3):
    mesh = plsc.VectorSubcoreMesh(core_axis_name="c", subcore_axis_name="s")

    @functools.partial(
        pl.kernel,
        mesh=mesh,
        out_type=jax.ShapeDtypeStruct((_B * _N, _D), jnp.float32),
        scratch_types=[
            pltpu.VMEM((_NCHUNKS, _C), jnp.int32),
            pltpu.VMEM((_NBUF, _C, _D), jnp.float32),
            pltpu.VMEM_SHARED((_NS, _NSLOT, _C, _D), jnp.float32),
        ]
        + [pltpu.SemaphoreType.DMA] * (_NBUF + 2 * _NSLOT),
    )
    def k(x_hbm, idx_hbm, out_hbm, idx_v, buf, shared, *sems):
        gsem = sems[:_NBUF]
        csem = sems[_NBUF : _NBUF + _NSLOT]
        wsem = sems[_NBUF + _NSLOT :]
        info = plsc.get_sparse_core_info()
        sid = lax.axis_index("s")
        wid = sid * info.num_cores + lax.axis_index("c")
        pltpu.sync_copy(idx_hbm.at[wid], idx_v)
        row_base = wid * _ROWS_PER_W

        def start_gather(c, b):
            pltpu.make_async_copy(x_hbm.at[idx_v.at[c]], buf.at[b], gsem[b]).start()

        def wait_gather(b):
            pltpu.make_async_copy(x_hbm.at[pl.ds(0, _C)], buf.at[b], gsem[b]).wait()

        def start_copy(b, s):
            pltpu.make_async_copy(buf.at[b], shared.at[sid].at[s], csem[s]).start()

        def wait_copy(b, s):
            pltpu.make_async_copy(buf.at[b], shared.at[sid].at[s], csem[s]).wait()

        def start_write(c, s):
            pltpu.make_async_copy(
                shared.at[sid].at[s], out_hbm.at[pl.ds(row_base + c * _C, _C)], wsem[s]
            ).start()

        def wait_write(s):
            pltpu.make_async_copy(
                shared.at[sid].at[s], out_hbm.at[pl.ds(row_base, _C)], wsem[s]
            ).wait()

        for p in range(_GDEPTH):
            start_gather(p, p)

        def body(g, carry):
            for b in range(_NBUF):
                c = _NBUF * g + b
                s = b % _NSLOT
                wait_gather(b)

                @pl.when(c >= _NSLOT)
                def _():
                    wait_write(s)

                start_copy(b, s)
                wait_copy(b, s)
                start_write(c, s)

                @pl.when(c + _GDEPTH < _NCHUNKS)
                def _():
                    start_gather(c + _GDEPTH, (b + _GDEPTH) % _NBUF)
            return carry

        lax.fori_loop(0, _NCHUNKS // _NBUF, body, 0)
        for s in range(_NSLOT):
            wait_write(s)

    return k(x_flat, idx3)


def _perm_indices(B, N):
    base_key = jax.random.key(42)

    def one(i):
        return jax.random.permutation(jax.random.fold_in(base_key, i), N)

    perm = jax.vmap(one)(jnp.arange(B))
    flat = perm.astype(jnp.int32) + (jnp.arange(B, dtype=jnp.int32) * N)[:, None]
    return flat.reshape(_NW, _NCHUNKS, _C)


def kernel(x):
    B, N, D = x.shape
    idx3 = _perm_indices(B, N)
    out = _gather(x.reshape(B * N, D), idx3)
    return out.reshape(B, N, D)
